# baseline (speedup 1.0000x reference)
import jax
import jax.numpy as jnp
from jax import lax
from jax.experimental import pallas as pl
from jax.experimental.pallas import tpu as pltpu, tpu_sc as plsc

_VOCAB = 1000000
_EMB = 32
_B = 4096
_L = 50
_NC = 2
_NS = 16
_NW = _NC * _NS
_TOTAL = _B * _L
_PER_W = _TOTAL // _NW
_CHUNK = 128
_NCH = _PER_W // _CHUNK

_mesh = plsc.VectorSubcoreMesh(
    core_axis_name="c", subcore_axis_name="s", num_cores=_NC, num_subcores=_NS
)


def _gather_body(idx_hbm, table_hbm, out_hbm, idx_v, rows_v, sem):
    wid = lax.axis_index("s") * _NC + lax.axis_index("c")
    base = wid * _PER_W
    pltpu.sync_copy(idx_hbm.at[wid], idx_v)

    def chunk(j, _):
        pltpu.async_copy(table_hbm.at[idx_v.at[j]], rows_v, sem).wait()
        pltpu.sync_copy(rows_v, out_hbm.at[pl.ds(base + j * _CHUNK, _CHUNK)])
        return 0

    lax.fori_loop(0, _NCH, chunk, 0)


_gather = pl.kernel(
    _gather_body,
    out_type=jax.ShapeDtypeStruct((_TOTAL, _EMB), jnp.float32),
    mesh=_mesh,
    scratch_types=[
        pltpu.VMEM((_NCH, _CHUNK), jnp.int32),
        pltpu.VMEM((_CHUNK, _EMB), jnp.float32),
        pltpu.SemaphoreType.DMA,
    ],
    compiler_params=pltpu.CompilerParams(use_tc_tiling_on_sc=False),
)


def kernel(unk_inputs, table):
    idx = jnp.transpose(unk_inputs).reshape(_NW, _NCH, _CHUNK)
    out = _gather(idx, table)
    return out.reshape(_L, _B, _EMB)


# speedup vs baseline: 1.0550x; 1.0550x over previous
import jax
import jax.numpy as jnp
from jax import lax
from jax.experimental import pallas as pl
from jax.experimental.pallas import tpu as pltpu, tpu_sc as plsc

_VOCAB = 1000000
_EMB = 32
_B = 4096
_L = 50
_NC = 2
_NS = 16
_NW = _NC * _NS
_TOTAL = _B * _L
_PER_W = _TOTAL // _NW
_CHUNK = 128
_NCH = _PER_W // _CHUNK

_mesh = plsc.VectorSubcoreMesh(
    core_axis_name="c", subcore_axis_name="s", num_cores=_NC, num_subcores=_NS
)


_SUP = 5
_SUPROWS = _SUP * _CHUNK
_NSUP = _PER_W // _SUPROWS


def _gather_body(idx_hbm, table_hbm, out_hbm, idx_v, rows_v, gsem0, gsem1):
    wid = lax.axis_index("s") * _NC + lax.axis_index("c")
    base = wid * _PER_W
    pltpu.sync_copy(idx_hbm.at[wid], idx_v)

    def _fire(s, b, sem):
        for j in range(_SUP):
            pltpu.async_copy(
                table_hbm.at[idx_v.at[s * _SUP + j]],
                rows_v.at[b, pl.ds(j * _CHUNK, _CHUNK)],
                sem,
            )

    def _drain(b, sem):
        pltpu.make_async_copy(
            table_hbm.at[pl.ds(0, _SUPROWS)], rows_v.at[b], sem
        ).wait()

    def _write(s, b):
        pltpu.sync_copy(
            rows_v.at[b], out_hbm.at[pl.ds(base + s * _SUPROWS, _SUPROWS)]
        )

    _fire(0, 0, gsem0)

    @pl.loop(0, _NSUP, step=2)
    def _loop(s0):
        _fire(s0 + 1, 1, gsem1)
        _drain(0, gsem0)
        _write(s0, 0)

        @pl.when(s0 + 2 < _NSUP)
        def _():
            _fire(s0 + 2, 0, gsem0)

        _drain(1, gsem1)
        _write(s0 + 1, 1)


_gather = pl.kernel(
    _gather_body,
    out_type=jax.ShapeDtypeStruct((_TOTAL, _EMB), jnp.float32),
    mesh=_mesh,
    scratch_types=[
        pltpu.VMEM((_NCH, _CHUNK), jnp.int32),
        pltpu.VMEM((2, _SUPROWS, _EMB), jnp.float32),
        pltpu.SemaphoreType.DMA,
        pltpu.SemaphoreType.DMA,
    ],
    compiler_params=pltpu.CompilerParams(use_tc_tiling_on_sc=False),
)


def kernel(unk_inputs, table):
    idx = jnp.transpose(unk_inputs).reshape(_NW, _NCH, _CHUNK)
    out = _gather(idx, table)
    return out.reshape(_L, _B, _EMB)
